# baseline (speedup 1.0000x reference)
import jax
import jax.numpy as jnp
from jax.experimental import pallas as pl

L1, L2, N, C = 2048, 4096, 8, 3
L2_BLK = 512


def _nn_kernel(c2_ref, c1_ref, out_ref):
    c2b = c2_ref[0]
    c1b = c1_ref[0]
    c2l = c2b.astype(jnp.bfloat16).astype(jnp.float32)
    c1l = c1b.astype(jnp.bfloat16).astype(jnp.float32)
    dots = c2l[:, 0:1] * c1l[0:1, :]
    dots = dots + c2l[:, 1:2] * c1l[1:2, :]
    dots = dots + c2l[:, 2:3] * c1l[2:3, :]
    sq1 = jnp.sum(c1b * c1b, axis=0, keepdims=True)
    sq2 = jnp.sum(c2b * c2b, axis=1, keepdims=True)
    d = (sq2 + sq1) - 2.0 * dots
    dmin = jnp.min(d, axis=-1, keepdims=True)
    iota = jax.lax.broadcasted_iota(jnp.int32, d.shape, 1)
    idx = jnp.min(jnp.where(d <= dmin, iota, L1), axis=-1)
    out_ref[0, 0, :] = idx


def kernel(coords1, coords2):
    l1, n, c = coords1.shape
    l2 = coords2.shape[0]
    c1t = jnp.transpose(coords1, (1, 2, 0))
    c2t = jnp.transpose(coords2, (1, 0, 2))

    out = pl.pallas_call(
        _nn_kernel,
        grid=(n, l2 // L2_BLK),
        in_specs=[
            pl.BlockSpec((1, L2_BLK, c), lambda i, j: (i, j, 0)),
            pl.BlockSpec((1, c, l1), lambda i, j: (i, 0, 0)),
        ],
        out_specs=pl.BlockSpec((1, 1, L2_BLK), lambda i, j: (i, 0, j)),
        out_shape=jax.ShapeDtypeStruct((n, 1, l2), jnp.int32),
    )(c2t, c1t)

    idx0 = out.reshape(n, l2).T.reshape(-1).astype(jnp.int64)
    idx1 = jnp.tile(jnp.arange(n, dtype=jnp.int64), l2)
    return idx0, idx1


# speedup vs baseline: 1.0556x; 1.0556x over previous
import jax
import jax.numpy as jnp
from jax.experimental import pallas as pl

L1, L2, N, C = 2048, 4096, 8, 3
L2_BLK = 512


def _nn_kernel(c2_ref, c1m2_ref, out_ref):
    c2b = c2_ref[0]
    c1m2 = c1m2_ref[0]
    dots2 = jax.lax.dot_general(
        c2b.astype(jnp.bfloat16), c1m2.astype(jnp.bfloat16),
        (((1,), (0,)), ((), ())),
        preferred_element_type=jnp.float32)
    sq1 = 0.25 * jnp.sum(c1m2 * c1m2, axis=0, keepdims=True)
    sq2 = jnp.sum(c2b * c2b, axis=1, keepdims=True)
    d = (sq2 + sq1) + dots2
    dmin = jnp.min(d, axis=-1, keepdims=True)
    iota = jax.lax.broadcasted_iota(jnp.int32, d.shape, 1)
    idx = jnp.min(jnp.where(d <= dmin, iota, L1), axis=-1)
    out_ref[0, 0, :] = idx


def kernel(coords1, coords2):
    l1, n, c = coords1.shape
    l2 = coords2.shape[0]
    c1t = jnp.transpose(coords1 * (-2.0), (1, 2, 0))
    c2t = jnp.transpose(coords2, (1, 0, 2))

    out = pl.pallas_call(
        _nn_kernel,
        grid=(n, l2 // L2_BLK),
        in_specs=[
            pl.BlockSpec((1, L2_BLK, c), lambda i, j: (i, j, 0)),
            pl.BlockSpec((1, c, l1), lambda i, j: (i, 0, 0)),
        ],
        out_specs=pl.BlockSpec((1, 1, L2_BLK), lambda i, j: (i, 0, j)),
        out_shape=jax.ShapeDtypeStruct((n, 1, l2), jnp.int32),
    )(c2t, c1t)

    idx0 = out.reshape(n, l2).T.reshape(-1).astype(jnp.int64)
    idx1 = jnp.tile(jnp.arange(n, dtype=jnp.int64), l2)
    return idx0, idx1


# speedup vs baseline: 1.3209x; 1.2514x over previous
import jax
import jax.numpy as jnp
from jax.experimental import pallas as pl

L1, L2, N, C = 2048, 4096, 8, 3
L2_BLK = 1024


def _nn_kernel(c2_ref, c1m2_ref, iota_ref, out_ref):
    c2b = c2_ref[0]
    c1m2 = c1m2_ref[0]
    dots2 = jax.lax.dot_general(
        c2b.astype(jnp.bfloat16), c1m2.astype(jnp.bfloat16),
        (((1,), (0,)), ((), ())),
        preferred_element_type=jnp.float32)
    sq1 = 0.25 * jnp.sum(c1m2 * c1m2, axis=0, keepdims=True)
    sq2 = jnp.sum(c2b * c2b, axis=1, keepdims=True)
    d = (sq2 + sq1) + dots2
    dmin = jnp.min(d, axis=-1, keepdims=True)
    iota = iota_ref[...]
    idx = jnp.min(jnp.where(d <= dmin, iota, float(L1)), axis=-1)
    out_ref[0, 0, :] = idx.astype(jnp.int32)


def kernel(coords1, coords2):
    l1, n, c = coords1.shape
    l2 = coords2.shape[0]
    c1t = jnp.transpose(coords1 * (-2.0), (1, 2, 0))
    c2t = jnp.transpose(coords2, (1, 0, 2))

    out = pl.pallas_call(
        _nn_kernel,
        grid=(n, l2 // L2_BLK),
        in_specs=[
            pl.BlockSpec((1, L2_BLK, c), lambda i, j: (i, j, 0)),
            pl.BlockSpec((1, c, l1), lambda i, j: (i, 0, 0)),
            pl.BlockSpec((1, l1), lambda i, j: (0, 0)),
        ],
        out_specs=pl.BlockSpec((1, 1, L2_BLK), lambda i, j: (i, 0, j)),
        out_shape=jax.ShapeDtypeStruct((n, 1, l2), jnp.int32),
    )(c2t, c1t, jnp.arange(l1, dtype=jnp.float32).reshape(1, l1))

    idx0 = out.reshape(n, l2).T.reshape(-1).astype(jnp.int64)
    idx1 = jnp.tile(jnp.arange(n, dtype=jnp.int64), l2)
    return idx0, idx1


# speedup vs baseline: 1.7294x; 1.3092x over previous
import jax
import jax.numpy as jnp
from jax.experimental import pallas as pl

L1, L2, N, C = 2048, 4096, 8, 3
L2_BLK = 1024


def _nn_kernel(c1m2_ref, c2_ref, iota_ref, out_ref):
    c1m2 = c1m2_ref[0]
    c2b = c2_ref[0]
    dots2 = jax.lax.dot_general(
        c1m2.astype(jnp.bfloat16), c2b.astype(jnp.bfloat16),
        (((1,), (0,)), ((), ())),
        preferred_element_type=jnp.float32)
    sq1 = 0.25 * jnp.sum(c1m2 * c1m2, axis=1, keepdims=True)
    sq2 = jnp.sum(c2b * c2b, axis=0, keepdims=True)
    d = (sq2 + sq1) + dots2
    dmin = jnp.min(d, axis=0, keepdims=True)
    iota = iota_ref[0]
    idx = jnp.min(jnp.where(d <= dmin, iota, float(L1)), axis=0)
    out_ref[0, 0, :] = idx.astype(jnp.int32)


def kernel(coords1, coords2):
    l1, n, c = coords1.shape
    l2 = coords2.shape[0]
    c1t = jnp.transpose(coords1 * (-2.0), (1, 0, 2))
    c2t = jnp.transpose(coords2, (1, 2, 0))
    iota = jnp.arange(l1, dtype=jnp.float32).reshape(1, l1, 1)

    out = pl.pallas_call(
        _nn_kernel,
        grid=(n, l2 // L2_BLK),
        in_specs=[
            pl.BlockSpec((1, l1, c), lambda i, j: (i, 0, 0)),
            pl.BlockSpec((1, c, L2_BLK), lambda i, j: (i, 0, j)),
            pl.BlockSpec((1, l1, 1), lambda i, j: (0, 0, 0)),
        ],
        out_specs=pl.BlockSpec((1, 1, L2_BLK), lambda i, j: (i, 0, j)),
        out_shape=jax.ShapeDtypeStruct((n, 1, l2), jnp.int32),
    )(c1t, c2t, iota)

    idx0 = out.reshape(n, l2).T.reshape(-1).astype(jnp.int64)
    idx1 = jnp.tile(jnp.arange(n, dtype=jnp.int64), l2)
    return idx0, idx1
